# baseline (speedup 1.0000x reference)
import jax
import jax.numpy as jnp
from jax import lax
from jax.experimental import pallas as pl
from jax.experimental.pallas import tpu as pltpu
from jax.experimental.pallas import tpu_sc as plsc

_R, _N = 64, 32768
_L = 16
_NCHUNK = _N // _L
_GRP = 16
_NGRP = _NCHUNK // _GRP
_NC, _NS = 2, 16
_NW = _NC * _NS
_ROWS_PER_W = _R // _NW
_NQ = 4
_QW = _N // _NQ
_QGRP = _NGRP // _NQ
_ZW = 8192
_NZ = _N // _ZW
_NEG = -3.0e38


def _chunkmax_quarter(row_v, cmax_v, q, gl):
    lane15 = lax.iota(jnp.int32, _L) == (_L - 1)
    nchq = _NCHUNK // _NQ

    @plsc.parallel_loop(q * nchq, (q + 1) * nchq, step=1, unroll=2,
                        carry=gl)
    def max_loop(ch, acc):
        v = row_v[pl.ds(ch * _L, _L)]
        cm = plsc.cummax(v)
        plsc.store_scatter(cmax_v, [jnp.full((_L,), ch, jnp.int32)], cm,
                           mask=lane15)
        return jnp.maximum(acc, v)

    return max_loop


def _row_tau(row_v, cmax_v, cnts_v, offs_v, colid_v, cand_v, gl):
    mx = jnp.max(gl)
    t0 = mx - 1.0
    lane0 = lax.iota(jnp.int32, _L) == 0

    @plsc.parallel_loop(0, _NGRP, step=1, unroll=2)
    def cnt_loop(g):
        m = cmax_v[pl.ds(g * _L, _L)] > t0
        cnt = plsc.all_reduce_population_count(m)
        plsc.store_scatter(cnts_v, [jnp.full((_L,), g, jnp.int32)], cnt,
                           mask=lane0)

    def scan_body(b, tot):
        c = cnts_v[pl.ds(b * _L, _L)]
        inc = plsc.cumsum(c)
        offs_v[pl.ds(b * _L, _L)] = inc - c + tot
        return tot + inc[_L - 1]

    wc = lax.fori_loop(0, _NGRP // _L, scan_body, jnp.int32(0))

    @plsc.parallel_loop(0, _NGRP, step=1, unroll=2)
    def colcap_loop(g):
        m = cmax_v[pl.ds(g * _L, _L)] > t0
        off = offs_v[pl.ds(g, _L)][0]
        ids = lax.iota(jnp.int32, _L) + g * _L
        plsc.store_compressed(colid_v.at[pl.ds(off, _L)], ids, mask=m)

    @plsc.parallel_loop(0, wc, step=1, unroll=2)
    def stage_loop(i):
        cid = colid_v[pl.ds(i, _L)][0]
        cand_v[pl.ds(i * _L, _L)] = row_v[pl.ds(cid * _L, _L)]

    cand_v[pl.ds(wc * _L, _L)] = jnp.full((_L,), _NEG, jnp.float32)
    npair = (wc + 1) >> 1

    def mich_cond(carry):
        tau, prev, it = carry
        return (tau > prev) & (it < jnp.int32(64))

    def mich_body(carry):
        tau, prev, it = carry

        def sbody(i, c4):
            s1, k1, s2, k2 = c4
            va = cand_v[pl.ds(i * (2 * _L), _L)]
            vb = cand_v[pl.ds(i * (2 * _L) + _L, _L)]
            ma = va > tau
            mb = vb > tau
            return (s1 + jnp.where(ma, va, 0.0), k1 + jnp.where(ma, 1, 0),
                    s2 + jnp.where(mb, vb, 0.0), k2 + jnp.where(mb, 1, 0))

        z_f = jnp.zeros((_L,), jnp.float32)
        z_i = jnp.zeros((_L,), jnp.int32)
        s1, k1, s2, k2 = lax.fori_loop(0, npair, sbody, (z_f, z_i, z_f, z_i))
        S = jnp.sum(s1 + s2)
        K = jnp.sum(k1 + k2)
        num = jnp.full((_L,), S - 1.0, jnp.float32)
        den = jnp.full((_L,), jnp.maximum(K, 1), jnp.int32).astype(jnp.float32)
        nt = jnp.max(num / den)
        return (nt, tau, it + 1)

    tau, _, _ = lax.while_loop(mich_cond, mich_body,
                               (t0, t0 - 1.0, jnp.int32(0)))
    return tau, wc


def _flush_row(colid_v, cand_v, out_row_hbm, tau, wc, sem):
    def flush_body(i, carry):
        cid = colid_v[pl.ds(i, _L)][0]
        v = cand_v[pl.ds(i * _L, _L)]
        cand_v[pl.ds(i * _L, _L)] = jnp.maximum(v - tau, 0.0)
        pltpu.async_copy(cand_v.at[pl.ds(i * _L, _L)],
                         out_row_hbm.at[pl.ds(cid * _L, _L)], sem)
        return carry

    lax.fori_loop(0, wc, flush_body, jnp.int32(0))

    def drain_body(i, carry):
        cid = colid_v[pl.ds(i, _L)][0]
        pltpu.make_async_copy(cand_v.at[pl.ds(i * _L, _L)],
                              out_row_hbm.at[pl.ds(cid * _L, _L)],
                              sem).wait()
        return carry

    lax.fori_loop(0, wc, drain_body, jnp.int32(0))


def _sparsemax_body(x_hbm, out_hbm, row_a, row_b, cmax_v, cnts_v, offs_v,
                    colid_v, cand_v, zero_v,
                    sem_a, sem_b, sem_za, sem_zb, sem_fa, sem_fb):
    c = lax.axis_index("c")
    s = lax.axis_index("s")
    wid = s * _NC + c
    r0 = wid * _ROWS_PER_W
    r1 = r0 + 1

    cps_a = [pltpu.async_copy(x_hbm.at[r0, pl.ds(q * _QW, _QW)],
                              row_a.at[pl.ds(q * _QW, _QW)], sem_a)
             for q in range(_NQ)]

    @plsc.parallel_loop(0, _ZW // _L, step=1, unroll=4)
    def zinit(i):
        zero_v[pl.ds(i * _L, _L)] = jnp.zeros((_L,), jnp.float32)

    gl = jnp.full((_L,), _NEG, jnp.float32)
    cp_b = None
    zf_a = zf_b = None
    for q in range(_NQ):
        cps_a[q].wait()
        if q == 0:
            cp_b = pltpu.async_copy(x_hbm.at[r1], row_b, sem_b)
            zf_a = [pltpu.async_copy(zero_v,
                                     out_hbm.at[r0, pl.ds(z * _ZW, _ZW)],
                                     sem_za)
                    for z in range(_NZ)]
            zf_b = [pltpu.async_copy(zero_v,
                                     out_hbm.at[r1, pl.ds(z * _ZW, _ZW)],
                                     sem_zb)
                    for z in range(_NZ)]
        gl = _chunkmax_quarter(row_a, cmax_v, q, gl)
    tau_a, wc_a = _row_tau(row_a, cmax_v, cnts_v, offs_v, colid_v, cand_v, gl)
    for cp in zf_a:
        cp.wait()
    _flush_row(colid_v, cand_v, out_hbm.at[r0], tau_a, wc_a, sem_fa)

    cp_b.wait()
    gl = jnp.full((_L,), _NEG, jnp.float32)
    for q in range(_NQ):
        gl = _chunkmax_quarter(row_b, cmax_v, q, gl)
    tau_b, wc_b = _row_tau(row_b, cmax_v, cnts_v, offs_v, colid_v, cand_v, gl)
    for cp in zf_b:
        cp.wait()
    _flush_row(colid_v, cand_v, out_hbm.at[r1], tau_b, wc_b, sem_fb)


def kernel(input):
    f = pl.kernel(
        _sparsemax_body,
        out_type=jax.ShapeDtypeStruct((_R, _N), jnp.float32),
        mesh=plsc.VectorSubcoreMesh(core_axis_name="c", subcore_axis_name="s"),
        compiler_params=pltpu.CompilerParams(needs_layout_passes=False,
                                             skip_device_barrier=True),
        scratch_types=[
            pltpu.VMEM((_N,), jnp.float32),
            pltpu.VMEM((_N,), jnp.float32),
            pltpu.VMEM((_NCHUNK,), jnp.float32),
            pltpu.VMEM((_NGRP,), jnp.int32),
            pltpu.VMEM((_NGRP + _L,), jnp.int32),
            pltpu.VMEM((_NCHUNK + _L,), jnp.int32),
            pltpu.VMEM(((_NCHUNK + 2) * _L,), jnp.float32),
            pltpu.VMEM((_ZW,), jnp.float32),
            pltpu.SemaphoreType.DMA,
            pltpu.SemaphoreType.DMA,
            pltpu.SemaphoreType.DMA,
            pltpu.SemaphoreType.DMA,
            pltpu.SemaphoreType.DMA,
            pltpu.SemaphoreType.DMA,
        ],
    )
    return f(input)


# speedup vs baseline: 1.1097x; 1.1097x over previous
import jax
import jax.numpy as jnp
from jax import lax
from jax.experimental import pallas as pl
from jax.experimental.pallas import tpu as pltpu
from jax.experimental.pallas import tpu_sc as plsc

_R, _N = 64, 32768
_L = 16
_NCHUNK = _N // _L
_GRP = 16
_NGRP = _NCHUNK // _GRP
_NCOL = _NGRP * _L
_NBLK = _NGRP // _L
_NC, _NS = 2, 16
_NW = _NC * _NS
_ROWS_PER_W = _R // _NW
_NQ = 4
_QW = _N // _NQ
_QGRP = _NGRP // _NQ
_NEG = -3.0e38


def _max_quarter(row_v, gmax_v, q, gl):
    @plsc.parallel_loop(q * _QGRP, (q + 1) * _QGRP, step=1, unroll=2,
                        carry=gl)
    def max_loop(g, acc):
        base = g * (_GRP * _L)
        vs = [row_v[pl.ds(base + u * _L, _L)] for u in range(_GRP)]
        while len(vs) > 1:
            vs = [jnp.maximum(vs[2 * i], vs[2 * i + 1])
                  for i in range(len(vs) // 2)]
        gmax_v[pl.ds(g * _L, _L)] = vs[0]
        return jnp.maximum(acc, vs[0])

    return max_loop


def _row_tau(row_v, gmax_v, cnts_v, offs_v, colid_v, colval_v, gl):
    mx = jnp.max(gl)
    t0 = mx - 1.0
    lane0 = lax.iota(jnp.int32, _L) == 0

    @plsc.parallel_loop(0, _NGRP, step=1, unroll=2)
    def cnt_loop(g):
        gv = gmax_v[pl.ds(g * _L, _L)]
        m = gv > t0
        cnt = plsc.all_reduce_population_count(m)
        plsc.store_scatter(cnts_v, [jnp.full((_L,), g, jnp.int32)], cnt,
                           mask=lane0)

    def scan_body(b, tot):
        c = cnts_v[pl.ds(b * _L, _L)]
        inc = plsc.cumsum(c)
        offs_v[pl.ds(b * _L, _L)] = inc - c + tot
        return tot + inc[_L - 1]

    wc = lax.fori_loop(0, _NBLK, scan_body, jnp.int32(0))

    @plsc.parallel_loop(0, _NGRP, step=1, unroll=2)
    def colcap_loop(g):
        gv = gmax_v[pl.ds(g * _L, _L)]
        m = gv > t0
        off = offs_v[pl.ds(g, _L)][0]
        ids = lax.iota(jnp.int32, _L) + g * _L
        plsc.store_compressed(colid_v.at[pl.ds(off, _L)], ids, mask=m)

    @plsc.parallel_loop(0, wc, step=1, unroll=2)
    def gather_loop(i):
        cid = colid_v[pl.ds(i, _L)][0]
        base = (cid >> 4) * (_GRP * _L) + (cid & (_L - 1))
        idx = base + lax.iota(jnp.int32, _L) * _L
        colval_v[pl.ds(i * _L, _L)] = plsc.load_gather(row_v, [idx])

    colval_v[pl.ds(wc * _L, _L)] = jnp.full((_L,), _NEG, jnp.float32)
    npair = (wc + 1) >> 1

    def mich_cond(carry):
        tau, prev, it = carry
        return (tau > prev) & (it < jnp.int32(64))

    def mich_body(carry):
        tau, prev, it = carry

        def sbody(i, c4):
            s1, k1, s2, k2 = c4
            va = colval_v[pl.ds(i * (2 * _L), _L)]
            vb = colval_v[pl.ds(i * (2 * _L) + _L, _L)]
            ma = va > tau
            mb = vb > tau
            return (s1 + jnp.where(ma, va, 0.0), k1 + jnp.where(ma, 1, 0),
                    s2 + jnp.where(mb, vb, 0.0), k2 + jnp.where(mb, 1, 0))

        z_f = jnp.zeros((_L,), jnp.float32)
        z_i = jnp.zeros((_L,), jnp.int32)
        s1, k1, s2, k2 = lax.fori_loop(0, npair, sbody, (z_f, z_i, z_f, z_i))
        S = jnp.sum(s1 + s2)
        K = jnp.sum(k1 + k2)
        num = jnp.full((_L,), S - 1.0, jnp.float32)
        den = jnp.full((_L,), jnp.maximum(K, 1), jnp.int32).astype(jnp.float32)
        nt = jnp.max(num / den)
        return (nt, tau, it + 1)

    tau, _, _ = lax.while_loop(mich_cond, mich_body,
                               (t0, t0 - 1.0, jnp.int32(0)))
    return tau


def _emit_output(row_v, out_row_hbm, tau, sem):
    copies = []
    for q in range(_NQ):

        @plsc.parallel_loop(q * _QGRP, (q + 1) * _QGRP, step=1, unroll=2)
        def out_loop(g):
            base = g * (_GRP * _L)
            for u in range(_GRP):
                sl = pl.ds(base + u * _L, _L)
                row_v[sl] = jnp.maximum(row_v[sl] - tau, 0.0)

        copies.append(pltpu.async_copy(
            row_v.at[pl.ds(q * _QW, _QW)],
            out_row_hbm.at[pl.ds(q * _QW, _QW)], sem))
    return copies


def _sparsemax_body(x_hbm, out_hbm, row_a, row_b, gmax_v, cnts_v, offs_v,
                    colid_v, colval_v, sem_a, sem_b, sem_oa, sem_ob):
    c = lax.axis_index("c")
    s = lax.axis_index("s")
    wid = s * _NC + c
    r0 = wid * _ROWS_PER_W
    r1 = r0 + 1

    cps_a = [pltpu.async_copy(x_hbm.at[r0, pl.ds(q * _QW, _QW)],
                              row_a.at[pl.ds(q * _QW, _QW)], sem_a)
             for q in range(_NQ)]

    gl = jnp.full((_L,), _NEG, jnp.float32)
    cp_b = None
    for q in range(_NQ):
        cps_a[q].wait()
        if q == 0:
            cp_b = pltpu.async_copy(x_hbm.at[r1], row_b, sem_b)
        gl = _max_quarter(row_a, gmax_v, q, gl)
    tau_a = _row_tau(row_a, gmax_v, cnts_v, offs_v, colid_v, colval_v, gl)
    out_a = _emit_output(row_a, out_hbm.at[r0], tau_a, sem_oa)

    cp_b.wait()
    gl = jnp.full((_L,), _NEG, jnp.float32)
    for q in range(_NQ):
        gl = _max_quarter(row_b, gmax_v, q, gl)
    tau_b = _row_tau(row_b, gmax_v, cnts_v, offs_v, colid_v, colval_v, gl)
    out_b = _emit_output(row_b, out_hbm.at[r1], tau_b, sem_ob)

    for cp in out_a + out_b:
        cp.wait()


def kernel(input):
    f = pl.kernel(
        _sparsemax_body,
        out_type=jax.ShapeDtypeStruct((_R, _N), jnp.float32),
        mesh=plsc.VectorSubcoreMesh(core_axis_name="c", subcore_axis_name="s"),
        compiler_params=pltpu.CompilerParams(needs_layout_passes=False,
                                             skip_device_barrier=True),
        scratch_types=[
            pltpu.VMEM((_N,), jnp.float32),
            pltpu.VMEM((_N,), jnp.float32),
            pltpu.VMEM((_NCOL,), jnp.float32),
            pltpu.VMEM((_NGRP,), jnp.int32),
            pltpu.VMEM((_NGRP + _L,), jnp.int32),
            pltpu.VMEM((_NCOL + _L,), jnp.int32),
            pltpu.VMEM((_N + 2 * _L,), jnp.float32),
            pltpu.SemaphoreType.DMA,
            pltpu.SemaphoreType.DMA,
            pltpu.SemaphoreType.DMA,
            pltpu.SemaphoreType.DMA,
        ],
    )
    return f(input)
